# baseline (speedup 1.0000x reference)
import functools

import jax
import jax.numpy as jnp
from jax import lax
from jax.experimental import pallas as pl
from jax.experimental.pallas import tpu as pltpu
from jax.experimental.pallas import tpu_sc as plsc

N = 10000
E = 320000
F = 128
G = 64
NLAYER = 3
C = 2

NC = 2
NS = 16
NW = NC * NS
CH = 80
NCHUNK = 128
EPW = NCHUNK * CH
EPAD = NW * EPW
NP = 10240
RPT = NP // NS

_mesh = plsc.VectorSubcoreMesh(core_axis_name="c", subcore_axis_name="s")


@functools.partial(
    pl.kernel,
    out_type=jax.ShapeDtypeStruct((NC, NP, F), jnp.float32),
    mesh=_mesh,
    scratch_types=[
        pltpu.VMEM((EPW,), jnp.int32),
        pltpu.VMEM((NCHUNK, CH), jnp.int32),
        pltpu.VMEM((CH, F), jnp.float32),
        pltpu.VMEM((CH, F), jnp.float32),
        pltpu.VMEM_SHARED((NP, F), jnp.float32),
        pltpu.SemaphoreType.DMA,
        pltpu.SemaphoreType.DMA,
    ],
)
def _segment_sum_sc(h_hbm, src_hbm, dst_hbm, zero_hbm, out_hbm,
                    sidx, didx, rows_a, rows_b, acc, sem_a, sem_b):
    c = lax.axis_index("c")
    s = lax.axis_index("s")
    w = s * NC + c
    pltpu.sync_copy(src_hbm.at[w], sidx)
    pltpu.sync_copy(dst_hbm.at[w], didx)
    pltpu.async_copy(h_hbm.at[sidx.at[pl.ds(0, CH)]], rows_a, sem_a)
    pltpu.async_copy(h_hbm.at[sidx.at[pl.ds(CH, CH)]], rows_b, sem_b)
    pltpu.sync_copy(zero_hbm.at[pl.ds(s * RPT, RPT)],
                    acc.at[pl.ds(s * RPT, RPT)])
    plsc.subcore_barrier()

    def pair(i, carry):
        a = 2 * i
        pltpu.make_async_copy(h_hbm.at[sidx.at[pl.ds(0, CH)]],
                              rows_a, sem_a).wait()
        pltpu.sync_copy(rows_a, acc.at[didx.at[a]], add=True)

        @pl.when(i < NCHUNK // 2 - 1)
        def _():
            pltpu.async_copy(h_hbm.at[sidx.at[pl.ds((a + 2) * CH, CH)]],
                             rows_a, sem_a)

        pltpu.make_async_copy(h_hbm.at[sidx.at[pl.ds(0, CH)]],
                              rows_b, sem_b).wait()
        pltpu.sync_copy(rows_b, acc.at[didx.at[a + 1]], add=True)

        @pl.when(i < NCHUNK // 2 - 1)
        def _():
            pltpu.async_copy(h_hbm.at[sidx.at[pl.ds((a + 3) * CH, CH)]],
                             rows_b, sem_b)

        return carry

    lax.fori_loop(0, NCHUNK // 2, pair, 0)
    plsc.subcore_barrier()
    pltpu.sync_copy(acc.at[pl.ds(s * RPT, RPT)],
                    out_hbm.at[c, pl.ds(s * RPT, RPT)])


def _bn(m, gamma, beta):
    mu = jnp.mean(m, axis=0, keepdims=True)
    var = jnp.mean(m * m, axis=0, keepdims=True) - mu * mu
    return (m - mu) * (gamma * lax.rsqrt(var + 1e-5)) + beta


def _gin_mlp(p_ref, h_ref, w1_ref, b1_ref, g1_ref, be1_ref,
             w2_ref, b2_ref, g2_ref, be2_ref, gp_ref, bp_ref):
    out = p_ref[0, :N, :] + p_ref[1, :N, :] + h_ref[...]
    m = jnp.dot(out, w1_ref[...], preferred_element_type=jnp.float32)
    m = _bn(m + b1_ref[...], g1_ref[...], be1_ref[...])
    m = jnp.dot(m, w2_ref[...], preferred_element_type=jnp.float32)
    m = _bn(m + b2_ref[...], g2_ref[...], be2_ref[...])
    m = _bn(m, gp_ref[...], bp_ref[...])
    return jnp.maximum(m, 0.0)


def _mlp_body(p_ref, h_ref, w1_ref, b1_ref, g1_ref, be1_ref,
              w2_ref, b2_ref, g2_ref, be2_ref, gp_ref, bp_ref, o_ref):
    o_ref[...] = _gin_mlp(p_ref, h_ref, w1_ref, b1_ref, g1_ref, be1_ref,
                          w2_ref, b2_ref, g2_ref, be2_ref, gp_ref, bp_ref)


def _mlp_pool_body(p_ref, h_ref, w1_ref, b1_ref, g1_ref, be1_ref,
                   w2_ref, b2_ref, g2_ref, be2_ref, gp_ref, bp_ref,
                   bidx_ref, wfc_ref, bfc_ref, o_ref, pool_scr):
    hf = _gin_mlp(p_ref, h_ref, w1_ref, b1_ref, g1_ref, be1_ref,
                  w2_ref, b2_ref, g2_ref, be2_ref, gp_ref, bp_ref)
    bidx = bidx_ref[...]

    def g_body(g, carry):
        vals = jnp.where(bidx == g, hf, -jnp.inf)
        pool_scr[pl.ds(g, 1), :] = jnp.max(vals, axis=0, keepdims=True)
        return carry

    lax.fori_loop(0, G, g_body, 0)
    o_ref[...] = (jnp.dot(pool_scr[...], wfc_ref[...],
                          preferred_element_type=jnp.float32) + bfc_ref[...])


def _mlp_layer(parts, h, w1, b1, g1, be1, w2, b2, g2, be2, gp, bp):
    return pl.pallas_call(
        _mlp_body,
        out_shape=jax.ShapeDtypeStruct((N, F), jnp.float32),
    )(parts, h, w1, b1, g1, be1, w2, b2, g2, be2, gp, bp)


def _mlp_pool_layer(parts, h, w1, b1, g1, be1, w2, b2, g2, be2, gp, bp,
                    bidx, wfc, bfc):
    return pl.pallas_call(
        _mlp_pool_body,
        out_shape=jax.ShapeDtypeStruct((G, F), jnp.float32),
        scratch_shapes=[pltpu.VMEM((G, F), jnp.float32)],
    )(parts, h, w1, b1, g1, be1, w2, b2, g2, be2, gp, bp, bidx, wfc, bfc)


def kernel(x, edge_index, batch, W1, b1, g1, be1, W2, b2, g2, be2,
           gp, bp, Wfc, bfc):
    npad = EPAD - E
    src3 = jnp.concatenate(
        [edge_index[0], jnp.zeros((npad,), jnp.int32)]).reshape(NW, EPW)
    dst3 = jnp.concatenate(
        [edge_index[1], jnp.full((npad,), N, jnp.int32)]).reshape(NW, NCHUNK, CH)
    zeros = jnp.zeros((NP, F), jnp.float32)
    bidx = batch.reshape(N, 1)
    wfc_pad = jnp.zeros((F, F), jnp.float32).at[:, :C].set(Wfc)
    bfc_pad = jnp.zeros((1, F), jnp.float32).at[0, :C].set(bfc)

    h = x
    for l in range(NLAYER):
        parts = _segment_sum_sc(h, src3, dst3, zeros)
        args = (parts, h,
                W1[l], b1[l].reshape(1, F), g1[l].reshape(1, F),
                be1[l].reshape(1, F), W2[l], b2[l].reshape(1, F),
                g2[l].reshape(1, F), be2[l].reshape(1, F),
                gp[l].reshape(1, F), bp[l].reshape(1, F))
        if l < NLAYER - 1:
            h = _mlp_layer(*args)
        else:
            logits_pad = _mlp_pool_layer(*args, bidx, wfc_pad, bfc_pad)
    return logits_pad[:, :C]


# speedup vs baseline: 1.9839x; 1.9839x over previous
import functools

import jax
import jax.numpy as jnp
from jax import lax
from jax.experimental import pallas as pl
from jax.experimental.pallas import tpu as pltpu
from jax.experimental.pallas import tpu_sc as plsc

N = 10000
E = 320000
F = 128
G = 64
NLAYER = 3
C = 2

NC = 2
NS = 16
NW = NC * NS
CH = 80
NCHUNK = 125
EPW = NCHUNK * CH
EPAD = NW * EPW
NP = 10240
RPT = NP // NS

_mesh = plsc.VectorSubcoreMesh(core_axis_name="c", subcore_axis_name="s")


@functools.partial(
    pl.kernel,
    out_type=jax.ShapeDtypeStruct((NC, NP, F), jnp.float32),
    mesh=_mesh,
    scratch_types=[
        pltpu.VMEM((NCHUNK, CH), jnp.int32),
        pltpu.VMEM((NCHUNK, CH), jnp.int32),
        pltpu.VMEM((CH, F), jnp.float32),
        pltpu.VMEM_SHARED((NP, F), jnp.float32),
        pltpu.SemaphoreType.DMA,
    ],
)
def _segment_sum_sc(h_hbm, src_hbm, dst_hbm, zero_hbm, out_hbm,
                    sidx, didx, rows, acc, sem):
    c = lax.axis_index("c")
    s = lax.axis_index("s")
    w = s * NC + c
    pltpu.sync_copy(src_hbm.at[w], sidx)
    pltpu.sync_copy(dst_hbm.at[w], didx)
    pltpu.sync_copy(zero_hbm.at[pl.ds(s * RPT, RPT)],
                    acc.at[pl.ds(s * RPT, RPT)])
    plsc.subcore_barrier()

    def chunk(j, carry):
        pltpu.async_copy(h_hbm.at[sidx.at[j]], rows, sem).wait()
        pltpu.sync_copy(rows, acc.at[didx.at[j]], add=True)
        return carry

    lax.fori_loop(0, NCHUNK, chunk, 0)
    plsc.subcore_barrier()
    pltpu.sync_copy(acc.at[pl.ds(s * RPT, RPT)],
                    out_hbm.at[c, pl.ds(s * RPT, RPT)])


def _bn(m, gamma, beta):
    mu = jnp.mean(m, axis=0, keepdims=True)
    var = jnp.mean(m * m, axis=0, keepdims=True) - mu * mu
    return (m - mu) * (gamma * lax.rsqrt(var + 1e-5)) + beta


def _gin_mlp(p_ref, h_ref, w1_ref, b1_ref, g1_ref, be1_ref,
             w2_ref, b2_ref, g2_ref, be2_ref, gp_ref, bp_ref):
    out = p_ref[0, :N, :] + p_ref[1, :N, :] + h_ref[...]
    m = jnp.dot(out, w1_ref[...], preferred_element_type=jnp.float32)
    m = _bn(m + b1_ref[...], g1_ref[...], be1_ref[...])
    m = jnp.dot(m, w2_ref[...], preferred_element_type=jnp.float32)
    m = _bn(m + b2_ref[...], g2_ref[...], be2_ref[...])
    m = _bn(m, gp_ref[...], bp_ref[...])
    return jnp.maximum(m, 0.0)


def _mlp_body(p_ref, h_ref, w1_ref, b1_ref, g1_ref, be1_ref,
              w2_ref, b2_ref, g2_ref, be2_ref, gp_ref, bp_ref, o_ref):
    o_ref[...] = _gin_mlp(p_ref, h_ref, w1_ref, b1_ref, g1_ref, be1_ref,
                          w2_ref, b2_ref, g2_ref, be2_ref, gp_ref, bp_ref)


def _mlp_pool_body(p_ref, h_ref, w1_ref, b1_ref, g1_ref, be1_ref,
                   w2_ref, b2_ref, g2_ref, be2_ref, gp_ref, bp_ref,
                   bidx_ref, wfc_ref, bfc_ref, o_ref, pool_scr):
    hf = _gin_mlp(p_ref, h_ref, w1_ref, b1_ref, g1_ref, be1_ref,
                  w2_ref, b2_ref, g2_ref, be2_ref, gp_ref, bp_ref)
    bidx = bidx_ref[...]

    def g_body(g, carry):
        vals = jnp.where(bidx == g, hf, -jnp.inf)
        pool_scr[pl.ds(g, 1), :] = jnp.max(vals, axis=0, keepdims=True)
        return carry

    lax.fori_loop(0, G, g_body, 0)
    o_ref[...] = (jnp.dot(pool_scr[...], wfc_ref[...],
                          preferred_element_type=jnp.float32) + bfc_ref[...])


def _mlp_layer(parts, h, w1, b1, g1, be1, w2, b2, g2, be2, gp, bp):
    return pl.pallas_call(
        _mlp_body,
        out_shape=jax.ShapeDtypeStruct((N, F), jnp.float32),
    )(parts, h, w1, b1, g1, be1, w2, b2, g2, be2, gp, bp)


def _mlp_pool_layer(parts, h, w1, b1, g1, be1, w2, b2, g2, be2, gp, bp,
                    bidx, wfc, bfc):
    return pl.pallas_call(
        _mlp_pool_body,
        out_shape=jax.ShapeDtypeStruct((G, F), jnp.float32),
        scratch_shapes=[pltpu.VMEM((G, F), jnp.float32)],
    )(parts, h, w1, b1, g1, be1, w2, b2, g2, be2, gp, bp, bidx, wfc, bfc)


def kernel(x, edge_index, batch, W1, b1, g1, be1, W2, b2, g2, be2,
           gp, bp, Wfc, bfc):
    src3 = edge_index[0].reshape(NW, NCHUNK, CH)
    dst3 = edge_index[1].reshape(NW, NCHUNK, CH)
    zeros = jnp.zeros((NP, F), jnp.float32)
    bidx = batch.reshape(N, 1)
    wfc_pad = jnp.zeros((F, F), jnp.float32).at[:, :C].set(Wfc)
    bfc_pad = jnp.zeros((1, F), jnp.float32).at[0, :C].set(bfc)

    h = x
    for l in range(NLAYER):
        parts = _segment_sum_sc(h, src3, dst3, zeros)
        args = (parts, h,
                W1[l], b1[l].reshape(1, F), g1[l].reshape(1, F),
                be1[l].reshape(1, F), W2[l], b2[l].reshape(1, F),
                g2[l].reshape(1, F), be2[l].reshape(1, F),
                gp[l].reshape(1, F), bp[l].reshape(1, F))
        if l < NLAYER - 1:
            h = _mlp_layer(*args)
        else:
            logits_pad = _mlp_pool_layer(*args, bidx, wfc_pad, bfc_pad)
    return logits_pad[:, :C]


# speedup vs baseline: 2.0772x; 1.0470x over previous
import functools

import jax
import jax.numpy as jnp
from jax import lax
from jax.experimental import pallas as pl
from jax.experimental.pallas import tpu as pltpu
from jax.experimental.pallas import tpu_sc as plsc

N = 10000
E = 320000
F = 128
G = 64
NLAYER = 3
C = 2

NC = 2
NS = 16
NW = NC * NS
CH = 80
NCHUNK = 125
EPW = NCHUNK * CH
EPAD = NW * EPW
NP = 10240
RPT = NP // NS

def _segment_sum_sc(h, src3, dst3, zeros):
    f = pl.kernel(
        _segsum_body,
        out_type=jax.ShapeDtypeStruct((NC, NP, F), jnp.float32),
        mesh=plsc.VectorSubcoreMesh(core_axis_name="c", subcore_axis_name="s"),
        scratch_types=[
            pltpu.VMEM((NCHUNK, CH), jnp.int32),
            pltpu.VMEM((NCHUNK, CH), jnp.int32),
            pltpu.VMEM((CH, F), jnp.float32),
            pltpu.VMEM_SHARED((NP, F), jnp.float32),
            pltpu.SemaphoreType.DMA,
        ],
    )
    return f(h, src3, dst3, zeros)


def _segsum_body(h_hbm, src_hbm, dst_hbm, zero_hbm, out_hbm,
                 sidx, didx, rows, acc, sem):
    c = lax.axis_index("c")
    s = lax.axis_index("s")
    w = s * NC + c
    pltpu.sync_copy(src_hbm.at[w], sidx)
    pltpu.sync_copy(dst_hbm.at[w], didx)
    pltpu.sync_copy(zero_hbm.at[pl.ds(s * RPT, RPT)],
                    acc.at[pl.ds(s * RPT, RPT)])
    plsc.subcore_barrier()

    def chunk(j, carry):
        pltpu.async_copy(h_hbm.at[sidx.at[j]], rows, sem).wait()
        pltpu.sync_copy(rows, acc.at[didx.at[j]], add=True)
        return carry

    lax.fori_loop(0, NCHUNK, chunk, 0)
    plsc.subcore_barrier()
    pltpu.sync_copy(acc.at[pl.ds(s * RPT, RPT)],
                    out_hbm.at[c, pl.ds(s * RPT, RPT)])


def _bn(m, gamma, beta):
    mu = jnp.mean(m, axis=0, keepdims=True)
    var = jnp.mean(m * m, axis=0, keepdims=True) - mu * mu
    return (m - mu) * (gamma * lax.rsqrt(var + 1e-5)) + beta


def _gin_mlp(p_ref, h_ref, w1_ref, b1_ref, g1_ref, be1_ref,
             w2_ref, b2_ref, g2_ref, be2_ref, gp_ref, bp_ref):
    out = p_ref[0, :N, :] + p_ref[1, :N, :] + h_ref[...]
    m = jnp.dot(out, w1_ref[...], preferred_element_type=jnp.float32)
    m = _bn(m + b1_ref[...], g1_ref[...], be1_ref[...])
    m = jnp.dot(m, w2_ref[...], preferred_element_type=jnp.float32)
    m = _bn(m + b2_ref[...], g2_ref[...], be2_ref[...])
    m = _bn(m, gp_ref[...], bp_ref[...])
    return jnp.maximum(m, 0.0)


def _mlp_body(p_ref, h_ref, w1_ref, b1_ref, g1_ref, be1_ref,
              w2_ref, b2_ref, g2_ref, be2_ref, gp_ref, bp_ref, o_ref):
    o_ref[...] = _gin_mlp(p_ref, h_ref, w1_ref, b1_ref, g1_ref, be1_ref,
                          w2_ref, b2_ref, g2_ref, be2_ref, gp_ref, bp_ref)


def _mlp_pool_body(p_ref, h_ref, w1_ref, b1_ref, g1_ref, be1_ref,
                   w2_ref, b2_ref, g2_ref, be2_ref, gp_ref, bp_ref,
                   bidx_ref, wfc_ref, bfc_ref, o_ref, pool_scr, hf_scr):
    hf = _gin_mlp(p_ref, h_ref, w1_ref, b1_ref, g1_ref, be1_ref,
                  w2_ref, b2_ref, g2_ref, be2_ref, gp_ref, bp_ref)
    hf_scr[...] = hf
    bidx = bidx_ref[...]

    def g_body(g, s0):
        s1 = jnp.sum((bidx <= g).astype(jnp.int32))
        nstrips = (s1 - s0 + 15) // 16

        def strip(k, acc):
            kc = jnp.minimum(s0 + 16 * k, N - 16)
            blk = hf_scr[pl.ds(kc, 16), :]
            rid = kc + lax.broadcasted_iota(jnp.int32, (16, 1), 0)
            ok = (rid >= s0) & (rid < s1)
            return jnp.maximum(acc, jnp.max(jnp.where(ok, blk, -jnp.inf),
                                            axis=0, keepdims=True))

        acc = lax.fori_loop(0, nstrips, strip,
                            jnp.full((1, F), -jnp.inf, jnp.float32))
        pool_scr[pl.ds(g, 1), :] = acc
        return s1

    lax.fori_loop(0, G, g_body, jnp.int32(0))
    o_ref[...] = (jnp.dot(pool_scr[...], wfc_ref[...],
                          preferred_element_type=jnp.float32) + bfc_ref[...])


def _mlp_layer(parts, h, w1, b1, g1, be1, w2, b2, g2, be2, gp, bp):
    return pl.pallas_call(
        _mlp_body,
        out_shape=jax.ShapeDtypeStruct((N, F), jnp.float32),
    )(parts, h, w1, b1, g1, be1, w2, b2, g2, be2, gp, bp)


def _mlp_pool_layer(parts, h, w1, b1, g1, be1, w2, b2, g2, be2, gp, bp,
                    bidx, wfc, bfc):
    return pl.pallas_call(
        _mlp_pool_body,
        out_shape=jax.ShapeDtypeStruct((G, F), jnp.float32),
        scratch_shapes=[pltpu.VMEM((G, F), jnp.float32),
                        pltpu.VMEM((N, F), jnp.float32)],
    )(parts, h, w1, b1, g1, be1, w2, b2, g2, be2, gp, bp, bidx, wfc, bfc)


def kernel(x, edge_index, batch, W1, b1, g1, be1, W2, b2, g2, be2,
           gp, bp, Wfc, bfc):
    src3 = edge_index[0].reshape(NW, NCHUNK, CH)
    dst3 = edge_index[1].reshape(NW, NCHUNK, CH)
    zeros = jnp.zeros((NP, F), jnp.float32)
    bidx = batch.reshape(N, 1)
    wfc_pad = jnp.zeros((F, F), jnp.float32).at[:, :C].set(Wfc)
    bfc_pad = jnp.zeros((1, F), jnp.float32).at[0, :C].set(bfc)

    h = x
    for l in range(NLAYER):
        parts = _segment_sum_sc(h, src3, dst3, zeros)
        args = (parts, h,
                W1[l], b1[l].reshape(1, F), g1[l].reshape(1, F),
                be1[l].reshape(1, F), W2[l], b2[l].reshape(1, F),
                g2[l].reshape(1, F), be2[l].reshape(1, F),
                gp[l].reshape(1, F), bp[l].reshape(1, F))
        if l < NLAYER - 1:
            h = _mlp_layer(*args)
        else:
            logits_pad = _mlp_pool_layer(*args, bidx, wfc_pad, bfc_pad)
    return logits_pad[:, :C]


# speedup vs baseline: 2.1048x; 1.0133x over previous
import jax
import jax.numpy as jnp
from jax import lax
from jax.experimental import pallas as pl
from jax.experimental.pallas import tpu as pltpu
from jax.experimental.pallas import tpu_sc as plsc

N = 10000
E = 320000
F = 128
G = 64
NLAYER = 3
C = 2

NC = 2
NS = 16
NW = NC * NS
CH = 80
NCHUNK = 125
EPW = NCHUNK * CH
EPAD = NW * EPW
NP = 10240
RPT = NP // NS

def _segment_sum_sc(h, src3, dst3, zeros):
    f = pl.kernel(
        _segsum_body,
        out_type=jax.ShapeDtypeStruct((NC, NP, F), jnp.float32),
        mesh=plsc.VectorSubcoreMesh(core_axis_name="c", subcore_axis_name="s"),
        scratch_types=[
            pltpu.VMEM((NCHUNK, CH), jnp.int32),
            pltpu.VMEM((NCHUNK, CH), jnp.int32),
            pltpu.VMEM((CH, F), jnp.float32),
            pltpu.VMEM_SHARED((NP, F), jnp.float32),
            pltpu.SemaphoreType.DMA,
        ],
    )
    return f(h, src3, dst3, zeros)


def _segsum_body(h_hbm, src_hbm, dst_hbm, zero_hbm, out_hbm,
                 sidx, didx, rows, acc, sem):
    c = lax.axis_index("c")
    s = lax.axis_index("s")
    w = s * NC + c
    pltpu.sync_copy(src_hbm.at[w], sidx)
    pltpu.sync_copy(dst_hbm.at[w], didx)
    pltpu.sync_copy(zero_hbm.at[pl.ds(s * RPT, RPT)],
                    acc.at[pl.ds(s * RPT, RPT)])
    plsc.subcore_barrier()

    def chunk(j, carry):
        pltpu.async_copy(h_hbm.at[sidx.at[j]], rows, sem).wait()
        pltpu.sync_copy(rows, acc.at[didx.at[j]], add=True)
        return carry

    lax.fori_loop(0, NCHUNK, chunk, 0)
    plsc.subcore_barrier()
    pltpu.sync_copy(acc.at[pl.ds(s * RPT, RPT)],
                    out_hbm.at[c, pl.ds(s * RPT, RPT)])


def _gin_mlp(p_ref, h_ref, w1_ref, b1_ref, g1_ref, be1_ref,
             w2_ref, b2_ref, g2_ref, be2_ref, gp_ref, bp_ref):
    out = p_ref[0, :N, :] + p_ref[1, :N, :] + h_ref[...]
    m1 = jnp.dot(out, w1_ref[...], preferred_element_type=jnp.float32)
    m1 = m1 + b1_ref[...]
    mu1 = jnp.mean(m1, axis=0, keepdims=True)
    v1 = jnp.mean(m1 * m1, axis=0, keepdims=True) - mu1 * mu1
    a1 = g1_ref[...] * lax.rsqrt(v1 + 1e-5)
    w2s = a1.reshape(F, 1) * w2_ref[...]
    c2 = jnp.dot(be1_ref[...] - mu1 * a1, w2_ref[...],
                 preferred_element_type=jnp.float32) + b2_ref[...]
    m2 = jnp.dot(m1, w2s, preferred_element_type=jnp.float32) + c2
    mu2 = jnp.mean(m2, axis=0, keepdims=True)
    v2 = jnp.mean(m2 * m2, axis=0, keepdims=True) - mu2 * mu2
    a2 = g2_ref[...] * lax.rsqrt(v2 + 1e-5)
    scale = a2 * gp_ref[...] * lax.rsqrt(a2 * a2 * v2 + 1e-5)
    return jnp.maximum((m2 - mu2) * scale + bp_ref[...], 0.0)


def _mlp_body(p_ref, h_ref, w1_ref, b1_ref, g1_ref, be1_ref,
              w2_ref, b2_ref, g2_ref, be2_ref, gp_ref, bp_ref, o_ref):
    o_ref[...] = _gin_mlp(p_ref, h_ref, w1_ref, b1_ref, g1_ref, be1_ref,
                          w2_ref, b2_ref, g2_ref, be2_ref, gp_ref, bp_ref)


def _mlp_pool_body(p_ref, h_ref, w1_ref, b1_ref, g1_ref, be1_ref,
                   w2_ref, b2_ref, g2_ref, be2_ref, gp_ref, bp_ref,
                   bidx_ref, wfc_ref, bfc_ref, o_ref, pool_scr, hf_scr):
    hf = _gin_mlp(p_ref, h_ref, w1_ref, b1_ref, g1_ref, be1_ref,
                  w2_ref, b2_ref, g2_ref, be2_ref, gp_ref, bp_ref)
    hf_scr[...] = hf
    bidx = bidx_ref[...]

    def g_body(g, s0):
        s1 = jnp.sum((bidx <= g).astype(jnp.int32))
        nstrips = (s1 - s0 + 31) // 32

        def strip(k, acc):
            kc = jnp.minimum(s0 + 32 * k, N - 32)
            blk = hf_scr[pl.ds(kc, 32), :]
            rid = kc + lax.broadcasted_iota(jnp.int32, (32, 1), 0)
            ok = (rid >= s0) & (rid < s1)
            return jnp.maximum(acc, jnp.max(jnp.where(ok, blk, -jnp.inf),
                                            axis=0, keepdims=True))

        acc = lax.fori_loop(0, nstrips, strip,
                            jnp.full((1, F), -jnp.inf, jnp.float32))
        pool_scr[pl.ds(g, 1), :] = acc
        return s1

    lax.fori_loop(0, G, g_body, jnp.int32(0))
    o_ref[...] = (jnp.dot(pool_scr[...], wfc_ref[...],
                          preferred_element_type=jnp.float32) + bfc_ref[...])


def _mlp_layer(parts, h, w1, b1, g1, be1, w2, b2, g2, be2, gp, bp):
    return pl.pallas_call(
        _mlp_body,
        out_shape=jax.ShapeDtypeStruct((N, F), jnp.float32),
    )(parts, h, w1, b1, g1, be1, w2, b2, g2, be2, gp, bp)


def _mlp_pool_layer(parts, h, w1, b1, g1, be1, w2, b2, g2, be2, gp, bp,
                    bidx, wfc, bfc):
    return pl.pallas_call(
        _mlp_pool_body,
        out_shape=jax.ShapeDtypeStruct((G, F), jnp.float32),
        scratch_shapes=[pltpu.VMEM((G, F), jnp.float32),
                        pltpu.VMEM((N, F), jnp.float32)],
    )(parts, h, w1, b1, g1, be1, w2, b2, g2, be2, gp, bp, bidx, wfc, bfc)


def kernel(x, edge_index, batch, W1, b1, g1, be1, W2, b2, g2, be2,
           gp, bp, Wfc, bfc):
    src3 = edge_index[0].reshape(NW, NCHUNK, CH)
    dst3 = edge_index[1].reshape(NW, NCHUNK, CH)
    zeros = jnp.zeros((NP, F), jnp.float32)
    bidx = batch.reshape(N, 1)
    wfc_pad = jnp.zeros((F, F), jnp.float32).at[:, :C].set(Wfc)
    bfc_pad = jnp.zeros((1, F), jnp.float32).at[0, :C].set(bfc)

    h = x
    for l in range(NLAYER):
        parts = _segment_sum_sc(h, src3, dst3, zeros)
        args = (parts, h,
                W1[l], b1[l].reshape(1, F), g1[l].reshape(1, F),
                be1[l].reshape(1, F), W2[l], b2[l].reshape(1, F),
                g2[l].reshape(1, F), be2[l].reshape(1, F),
                gp[l].reshape(1, F), bp[l].reshape(1, F))
        if l < NLAYER - 1:
            h = _mlp_layer(*args)
        else:
            logits_pad = _mlp_pool_layer(*args, bidx, wfc_pad, bfc_pad)
    return logits_pad[:, :C]
